# baseline (speedup 1.0000x reference)
import functools

import numpy as np
import jax
import jax.numpy as jnp
from jax import lax
from jax.experimental import pallas as pl
from jax.experimental.pallas import tpu as pltpu
from jax.experimental.pallas import tpu_sc as plsc

NUM_LEVELS = 16
LEVEL_DIM = 2
BASE_RES = 16
LOG2_T = 19
T = 2 ** LOG2_T
N_PTS = 262144
DESIRED_RES = 2048
PER_LEVEL_SCALE = float(np.exp2(np.log2(DESIRED_RES / BASE_RES) / (NUM_LEVELS - 1)))
RES = [int(np.floor(BASE_RES * PER_LEVEL_SCALE ** l)) for l in range(NUM_LEVELS)]
P1 = np.int32(np.uint32(2654435761).astype(np.int32))
P2 = np.int32(805459861)
MASK = np.int32(T - 1)

NC, NS = 2, 16
N_TILES = NC * NS
NPT = N_PTS // N_TILES
G = 16
N_GROUPS = NPT // G

ENC_DIM = NUM_LEVELS * LEVEL_DIM

N_DENSE = 2
SIDE = [RES[l] + 2 for l in range(N_DENSE)]
DCELLS = [SIDE[l] ** 3 for l in range(N_DENSE)]
DBASE = [0, 2 * DCELLS[0]]
DWORDS = 2 * sum(DCELLS)


def _make_sc_compiler_params():
    import dataclasses
    cp = pltpu.CompilerParams()
    if "needs_layout_passes" in pltpu.CompilerParams.__dataclass_fields__:
        cp = dataclasses.replace(cp, needs_layout_passes=False)
    if "use_tc_tiling_on_sc" in pltpu.CompilerParams.__dataclass_fields__:
        cp = dataclasses.replace(cp, use_tc_tiling_on_sc=False)
    return cp


@functools.partial(
    pl.kernel,
    out_type=jax.ShapeDtypeStruct((ENC_DIM, N_PTS), jnp.float32),
    mesh=plsc.VectorSubcoreMesh(core_axis_name="c", subcore_axis_name="s"),
    compiler_params=_make_sc_compiler_params(),
    scratch_types=[
        pltpu.VMEM((3, NPT), jnp.float32),
        pltpu.VMEM((2, NUM_LEVELS, 128), jnp.int32),
        pltpu.VMEM((2, NUM_LEVELS, 128), jnp.int32),
        pltpu.VMEM((2, NUM_LEVELS, 128), jnp.float32),
        pltpu.VMEM((2, NUM_LEVELS, 128, 8), jnp.float32),
        pltpu.VMEM((ENC_DIM, 128), jnp.float32),
        pltpu.VMEM((DWORDS,), jnp.float32),
        pltpu.VMEM((128,), jnp.int32),
        pltpu.VMEM((128, 8), jnp.float32),
        pltpu.SemaphoreType.DMA,
        pltpu.SemaphoreType.DMA,
    ],
)
def _encode(xT_hbm, tab_hbm, enc_hbm, xv, idx_v, off_v, w_v, rows_v, enc_v,
            dense_v, bidx_v, brows_v, sem0, sem1):
    wid = lax.axis_index("s") * NC + lax.axis_index("c")
    base_pt = wid * NPT
    pltpu.sync_copy(xT_hbm.at[:, pl.ds(base_pt, NPT)], xv)

    iota = lax.iota(jnp.int32, 16)
    one_f = jnp.zeros((16,), jnp.float32) + 1.0
    sems = (sem0, sem1)

    for l in range(N_DENSE):
        side = np.int32(SIDE[l])
        side2 = np.int32(SIDE[l] * SIDE[l])
        rbase = np.int32(l * (T // 4))
        nbatch = (DCELLS[l] + 127) // 128
        ncells = np.int32(DCELLS[l])

        @pl.loop(0, nbatch)
        def _build(bi, l=l, side=side, side2=side2, rbase=rbase, ncells=ncells):
            base_cell = bi * 128
            for u in range(8):
                cid0 = base_cell + np.int32(u * 16)
                cid = jnp.minimum(cid0 + iota, ncells - 1)
                ix = cid // side2
                rem1 = cid - ix * side2
                iy = rem1 // side
                iz = rem1 - iy * side
                h = (ix ^ (iy * P1) ^ (iz * P2)) & MASK
                bidx_v[pl.ds(u * 16, 16)] = lax.shift_right_logical(h, 2) + rbase
            pltpu.async_copy(tab_hbm.at[bidx_v], brows_v, sem0).wait()
            for u in range(8):
                cid0 = base_cell + np.int32(u * 16)
                cid = jnp.minimum(cid0 + iota, ncells - 1)
                ix = cid // side2
                rem1 = cid - ix * side2
                iy = rem1 // side
                iz = rem1 - iy * side
                h = (ix ^ (iy * P1) ^ (iz * P2)) & MASK
                off = lax.shift_left(h & 3, 1)
                ridx = iota + np.int32(u * 16)
                v0 = plsc.load_gather(brows_v, [ridx, off])
                v1 = plsc.load_gather(brows_v, [ridx, off + 1])
                dst = cid * 2 + np.int32(DBASE[l])
                plsc.store_scatter(dense_v, [dst], v0)
                plsc.store_scatter(dense_v, [dst + 1], v1)

    def phase1(g, b):
        lx = g * G
        px = xv[0, pl.ds(lx, G)]
        py = xv[1, pl.ds(lx, G)]
        pz = xv[2, pl.ds(lx, G)]
        x01x = (px + 1.0) / 2.0
        x01y = (py + 1.0) / 2.0
        x01z = (pz + 1.0) / 2.0
        ib = idx_v.at[b]
        ob = off_v.at[b]
        wb = w_v.at[b]
        for l in range(NUM_LEVELS):
            resf = np.float32(RES[l])
            posx = x01x * resf
            posy = x01y * resf
            posz = x01z * resf
            ix = posx.astype(jnp.int32)
            iy = posy.astype(jnp.int32)
            iz = posz.astype(jnp.int32)
            fx = posx - ix.astype(jnp.float32)
            fy = posy - iy.astype(jnp.float32)
            fz = posz - iz.astype(jnp.float32)
            a0 = ix
            a1 = a0 + 1
            b0 = iy * P1
            b1 = b0 + P1
            c0 = iz * P2
            c1 = c0 + P2
            rbase = np.int32(l * (T // 4))
            wx0 = one_f - fx
            wy0 = one_f - fy
            wz0 = one_f - fz
            k = 0
            if l < N_DENSE:
                sde = np.int32(SIDE[l])
                dy0 = a0 * sde + iy
                dy1 = dy0 + sde
                dbse = np.int32(DBASE[l])
                for cx, dyv in ((0, dy0), (1, dy1)):
                    wxc = fx if cx else wx0
                    for cy in (0, 1):
                        wxy = wxc * (fy if cy else wy0)
                        dz = (dyv + cy) * sde + iz
                        for cz in (0, 1):
                            ob[l, pl.ds(k * 16, 16)] = (dz + cz) * 2 + dbse
                            wb[l, pl.ds(k * 16, 16)] = wxy * (fz if cz else wz0)
                            k += 1
                continue
            for cx, av in ((0, a0), (1, a1)):
                wxc = fx if cx else wx0
                for cy, bv in ((0, b0), (1, b1)):
                    wxy = wxc * (fy if cy else wy0)
                    ab = av ^ bv
                    for cz, cv in ((0, c0), (1, c1)):
                        h = (ab ^ cv) & MASK
                        ib[l, pl.ds(k * 16, 16)] = lax.shift_right_logical(h, 2) + rbase
                        ob[l, pl.ds(k * 16, 16)] = lax.shift_left(h & 3, 1)
                        wb[l, pl.ds(k * 16, 16)] = wxy * (fz if cz else wz0)
                        k += 1
        for l in range(N_DENSE, NUM_LEVELS):
            pltpu.async_copy(tab_hbm.at[idx_v.at[b].at[l]],
                             rows_v.at[b].at[l], sems[b])

    def wait(b):
        for l in range(N_DENSE, NUM_LEVELS):
            pltpu.make_async_copy(tab_hbm.at[idx_v.at[b].at[l]],
                                  rows_v.at[b].at[l], sems[b]).wait()

    def phase3(g, b):
        col = (g % 8) * G
        for l in range(NUM_LEVELS):
            f0 = jnp.zeros((16,), jnp.float32)
            f1 = jnp.zeros((16,), jnp.float32)
            rl = rows_v.at[b].at[l]
            for c in range(8):
                w = w_v[b, l, pl.ds(c * 16, 16)]
                off = off_v[b, l, pl.ds(c * 16, 16)]
                if l < N_DENSE:
                    v0 = plsc.load_gather(dense_v, [off])
                    v1 = plsc.load_gather(dense_v, [off + 1])
                else:
                    ridx = iota + np.int32(c * 16)
                    v0 = plsc.load_gather(rl, [ridx, off])
                    v1 = plsc.load_gather(rl, [ridx, off + 1])
                f0 = f0 + w * v0
                f1 = f1 + w * v1
            enc_v[2 * l, pl.ds(col, G)] = f0
            enc_v[2 * l + 1, pl.ds(col, G)] = f1

        @pl.when(g % 8 == 7)
        def _flush():
            o = pl.multiple_of(base_pt + (g - 7) * G, 128)
            pltpu.sync_copy(enc_v, enc_hbm.at[:, pl.ds(o, 128)])

    phase1(0, 0)

    @pl.loop(0, N_GROUPS // 2)
    def _pair(j):
        g0 = j * 2
        phase1(g0 + 1, 1)
        wait(0)
        phase3(g0, 0)

        @pl.when(j < N_GROUPS // 2 - 1)
        def _():
            phase1(g0 + 2, 0)

        wait(1)
        phase3(g0 + 1, 1)


def _mlp_body(enc_ref, w0t_ref, w1t_ref, w2_ref, out_ref):
    e = enc_ref[...]
    h = jnp.maximum(jnp.dot(w0t_ref[...], e, preferred_element_type=jnp.float32), 0.0)
    h = jnp.maximum(jnp.dot(w1t_ref[...], h, preferred_element_type=jnp.float32), 0.0)
    out_ref[...] = jnp.sum(h * w2_ref[...], axis=0, keepdims=True)


BN = 4096


def _mlp(encT, W0T, W1T, w2):
    return pl.pallas_call(
        _mlp_body,
        grid=(N_PTS // BN,),
        in_specs=[
            pl.BlockSpec((ENC_DIM, BN), lambda i: (0, i)),
            pl.BlockSpec((64, ENC_DIM), lambda i: (0, 0)),
            pl.BlockSpec((64, 64), lambda i: (0, 0)),
            pl.BlockSpec((64, 1), lambda i: (0, 0)),
        ],
        out_specs=pl.BlockSpec((1, BN), lambda i: (0, i)),
        out_shape=jax.ShapeDtypeStruct((1, N_PTS), jnp.float32),
    )(encT, W0T, W1T, w2)


def _make_ileave_mats():
    ev = np.zeros((256, 128), np.float32)
    od = np.zeros((256, 128), np.float32)
    for j in range(64):
        ev[j, 2 * j] = 1.0
        ev[128 + j, 2 * j + 1] = 1.0
        od[64 + j, 2 * j] = 1.0
        od[192 + j, 2 * j + 1] = 1.0
    return ev, od


_EV, _OD = _make_ileave_mats()


def _ileave_body(in_ref, ev_ref, od_ref, out_ref):
    x = in_ref[...].reshape(128, 2, 128)
    ab = jnp.concatenate([x[:, 0, :], x[:, 1, :]], axis=1)
    abh = ab.astype(jnp.bfloat16).astype(jnp.float32)
    abl = ab - abh
    ev = ev_ref[...]
    od = od_ref[...]
    d = jax.lax.Precision.DEFAULT
    evens = jnp.dot(abh, ev, precision=d) + jnp.dot(abl, ev, precision=d)
    odds = jnp.dot(abh, od, precision=d) + jnp.dot(abl, od, precision=d)
    out_ref[...] = jnp.stack([evens, odds], axis=1).reshape(256, 128)


def _interleave(t128):
    full = lambda i: (0, 0)
    return pl.pallas_call(
        _ileave_body,
        grid=(131072 // 256,),
        in_specs=[
            pl.BlockSpec((256, 128), lambda i: (i, 0)),
            pl.BlockSpec((256, 128), full),
            pl.BlockSpec((256, 128), full),
        ],
        out_specs=pl.BlockSpec((256, 128), lambda i: (i, 0)),
        out_shape=jax.ShapeDtypeStruct((131072, 128), jnp.float32),
    )(t128, jnp.asarray(_EV), jnp.asarray(_OD))


def kernel(x, table, W0, W1, W2):
    xT = x.T
    t128 = table.reshape(NUM_LEVELS, T // 128, 128, LEVEL_DIM)
    t128 = t128.transpose(0, 1, 3, 2).reshape(131072, 128)
    tab = _interleave(t128).reshape(NUM_LEVELS * T * LEVEL_DIM // 8, 8)
    encT = _encode(xT, tab)
    sig = _mlp(encT, W0.T, W1.T, W2[:, 0:1])
    return sig.reshape(N_PTS)


# speedup vs baseline: 1.0599x; 1.0599x over previous
import functools

import numpy as np
import jax
import jax.numpy as jnp
from jax import lax
from jax.experimental import pallas as pl
from jax.experimental.pallas import tpu as pltpu
from jax.experimental.pallas import tpu_sc as plsc

NUM_LEVELS = 16
LEVEL_DIM = 2
BASE_RES = 16
LOG2_T = 19
T = 2 ** LOG2_T
N_PTS = 262144
DESIRED_RES = 2048
PER_LEVEL_SCALE = float(np.exp2(np.log2(DESIRED_RES / BASE_RES) / (NUM_LEVELS - 1)))
RES = [int(np.floor(BASE_RES * PER_LEVEL_SCALE ** l)) for l in range(NUM_LEVELS)]
P1 = np.int32(np.uint32(2654435761).astype(np.int32))
P2 = np.int32(805459861)
MASK = np.int32(T - 1)

NC, NS = 2, 16
N_TILES = NC * NS
NPT = N_PTS // N_TILES
G = 16
N_GROUPS = NPT // G

ENC_DIM = NUM_LEVELS * LEVEL_DIM


def _make_sc_compiler_params():
    import dataclasses
    cp = pltpu.CompilerParams()
    if "needs_layout_passes" in pltpu.CompilerParams.__dataclass_fields__:
        cp = dataclasses.replace(cp, needs_layout_passes=False)
    if "use_tc_tiling_on_sc" in pltpu.CompilerParams.__dataclass_fields__:
        cp = dataclasses.replace(cp, use_tc_tiling_on_sc=False)
    return cp


@functools.partial(
    pl.kernel,
    out_type=jax.ShapeDtypeStruct((ENC_DIM, N_PTS), jnp.float32),
    mesh=plsc.VectorSubcoreMesh(core_axis_name="c", subcore_axis_name="s"),
    compiler_params=_make_sc_compiler_params(),
    scratch_types=[
        pltpu.VMEM((3, NPT), jnp.float32),
        pltpu.VMEM((2, NUM_LEVELS, 128), jnp.int32),
        pltpu.VMEM((2, NUM_LEVELS, 128), jnp.int32),
        pltpu.VMEM((2, NUM_LEVELS, 128), jnp.float32),
        pltpu.VMEM((2, NUM_LEVELS, 128, 8), jnp.float32),
        pltpu.VMEM((ENC_DIM, 128), jnp.float32),
        pltpu.SemaphoreType.DMA,
        pltpu.SemaphoreType.DMA,
    ],
)
def _encode(xT_hbm, tab_hbm, enc_hbm, xv, idx_v, off_v, w_v, rows_v, enc_v,
            sem0, sem1):
    wid = lax.axis_index("s") * NC + lax.axis_index("c")
    base_pt = wid * NPT
    pltpu.sync_copy(xT_hbm.at[:, pl.ds(base_pt, NPT)], xv)

    iota = lax.iota(jnp.int32, 16)
    one_f = jnp.zeros((16,), jnp.float32) + 1.0
    sems = (sem0, sem1)

    def phase1(g, b):
        lx = g * G
        px = xv[0, pl.ds(lx, G)]
        py = xv[1, pl.ds(lx, G)]
        pz = xv[2, pl.ds(lx, G)]
        x01x = (px + 1.0) / 2.0
        x01y = (py + 1.0) / 2.0
        x01z = (pz + 1.0) / 2.0
        ib = idx_v.at[b]
        ob = off_v.at[b]
        wb = w_v.at[b]
        for l in range(NUM_LEVELS):
            resf = np.float32(RES[l])
            posx = x01x * resf
            posy = x01y * resf
            posz = x01z * resf
            ix = posx.astype(jnp.int32)
            iy = posy.astype(jnp.int32)
            iz = posz.astype(jnp.int32)
            fx = posx - ix.astype(jnp.float32)
            fy = posy - iy.astype(jnp.float32)
            fz = posz - iz.astype(jnp.float32)
            a0 = ix
            a1 = a0 + 1
            b0 = iy * P1
            b1 = b0 + P1
            c0 = iz * P2
            c1 = c0 + P2
            rbase = np.int32(l * (T // 4))
            wx0 = one_f - fx
            wy0 = one_f - fy
            wz0 = one_f - fz
            k = 0
            for cx, av in ((0, a0), (1, a1)):
                wxc = fx if cx else wx0
                for cy, bv in ((0, b0), (1, b1)):
                    wxy = wxc * (fy if cy else wy0)
                    ab = av ^ bv
                    for cz, cv in ((0, c0), (1, c1)):
                        h = (ab ^ cv) & MASK
                        ib[l, pl.ds(k * 16, 16)] = lax.shift_right_logical(h, 2) + rbase
                        ob[l, pl.ds(k * 16, 16)] = lax.shift_left(h & 3, 1)
                        wb[l, pl.ds(k * 16, 16)] = wxy * (fz if cz else wz0)
                        k += 1
        for l in range(NUM_LEVELS):
            pltpu.async_copy(tab_hbm.at[idx_v.at[b].at[l]],
                             rows_v.at[b].at[l], sems[b])

    def wait(b):
        for l in range(NUM_LEVELS):
            pltpu.make_async_copy(tab_hbm.at[idx_v.at[b].at[l]],
                                  rows_v.at[b].at[l], sems[b]).wait()

    def phase3(g, b):
        col = (g % 8) * G
        for l in range(NUM_LEVELS):
            f0 = jnp.zeros((16,), jnp.float32)
            f1 = jnp.zeros((16,), jnp.float32)
            rl = rows_v.at[b].at[l]
            for c in range(8):
                ridx = iota + np.int32(c * 16)
                w = w_v[b, l, pl.ds(c * 16, 16)]
                off = off_v[b, l, pl.ds(c * 16, 16)]
                v0 = plsc.load_gather(rl, [ridx, off])
                v1 = plsc.load_gather(rl, [ridx, off + 1])
                f0 = f0 + w * v0
                f1 = f1 + w * v1
            enc_v[2 * l, pl.ds(col, G)] = f0
            enc_v[2 * l + 1, pl.ds(col, G)] = f1

        @pl.when(g % 8 == 7)
        def _flush():
            o = pl.multiple_of(base_pt + (g - 7) * G, 128)
            pltpu.sync_copy(enc_v, enc_hbm.at[:, pl.ds(o, 128)])

    phase1(0, 0)

    @pl.loop(0, N_GROUPS // 2)
    def _pair(j):
        g0 = j * 2
        phase1(g0 + 1, 1)
        wait(0)
        phase3(g0, 0)

        @pl.when(j < N_GROUPS // 2 - 1)
        def _():
            phase1(g0 + 2, 0)

        wait(1)
        phase3(g0 + 1, 1)


def _mlp_body(enc_ref, w0t_ref, w1t_ref, w2_ref, out_ref):
    e = enc_ref[...]
    h = jnp.maximum(jnp.dot(w0t_ref[...], e, preferred_element_type=jnp.float32), 0.0)
    h = jnp.maximum(jnp.dot(w1t_ref[...], h, preferred_element_type=jnp.float32), 0.0)
    out_ref[...] = jnp.sum(h * w2_ref[...], axis=0, keepdims=True)


BN = 4096


def _mlp(encT, W0T, W1T, w2):
    return pl.pallas_call(
        _mlp_body,
        grid=(N_PTS // BN,),
        in_specs=[
            pl.BlockSpec((ENC_DIM, BN), lambda i: (0, i)),
            pl.BlockSpec((64, ENC_DIM), lambda i: (0, 0)),
            pl.BlockSpec((64, 64), lambda i: (0, 0)),
            pl.BlockSpec((64, 1), lambda i: (0, 0)),
        ],
        out_specs=pl.BlockSpec((1, BN), lambda i: (0, i)),
        out_shape=jax.ShapeDtypeStruct((1, N_PTS), jnp.float32),
    )(encT, W0T, W1T, w2)


def _make_ileave_mats():
    ev = np.zeros((256, 128), np.float32)
    od = np.zeros((256, 128), np.float32)
    for j in range(64):
        ev[j, 2 * j] = 1.0
        ev[128 + j, 2 * j + 1] = 1.0
        od[64 + j, 2 * j] = 1.0
        od[192 + j, 2 * j + 1] = 1.0
    return ev, od


_EV, _OD = _make_ileave_mats()


def _ileave_body(in_ref, ev_ref, od_ref, out_ref):
    x = in_ref[...].reshape(128, 2, 128)
    ab = jnp.concatenate([x[:, 0, :], x[:, 1, :]], axis=1)
    abh = ab.astype(jnp.bfloat16).astype(jnp.float32)
    abl = ab - abh
    ev = ev_ref[...]
    od = od_ref[...]
    d = jax.lax.Precision.DEFAULT
    evens = jnp.dot(abh, ev, precision=d) + jnp.dot(abl, ev, precision=d)
    odds = jnp.dot(abh, od, precision=d) + jnp.dot(abl, od, precision=d)
    out_ref[...] = jnp.stack([evens, odds], axis=1).reshape(256, 128)


def _interleave(t128):
    full = lambda i: (0, 0)
    return pl.pallas_call(
        _ileave_body,
        grid=(131072 // 256,),
        in_specs=[
            pl.BlockSpec((256, 128), lambda i: (i, 0)),
            pl.BlockSpec((256, 128), full),
            pl.BlockSpec((256, 128), full),
        ],
        out_specs=pl.BlockSpec((256, 128), lambda i: (i, 0)),
        out_shape=jax.ShapeDtypeStruct((131072, 128), jnp.float32),
    )(t128, jnp.asarray(_EV), jnp.asarray(_OD))


def kernel(x, table, W0, W1, W2):
    xT = x.T
    t128 = table.reshape(NUM_LEVELS, T // 128, 128, LEVEL_DIM)
    t128 = t128.transpose(0, 1, 3, 2).reshape(131072, 128)
    tab = _interleave(t128).reshape(NUM_LEVELS * T * LEVEL_DIM // 8, 8)
    encT = _encode(xT, tab)
    sig = _mlp(encT, W0.T, W1.T, W2[:, 0:1])
    return sig.reshape(N_PTS)
